# Initial kernel scaffold; baseline (speedup 1.0000x reference)
import jax
import jax.numpy as jnp
from jax.experimental import pallas as pl


def kernel(y, midi_notes, F):
    raise NotImplementedError("write your pallas kernel here")


# speedup vs baseline: 3.3905x; 3.3905x over previous
import functools

import jax
import jax.numpy as jnp
from jax import lax
from jax.experimental import pallas as pl
from jax.experimental.pallas import tpu as pltpu
from jax.experimental.pallas import tpu_sc as plsc

B, N, T, F_BINS = 16, 128, 4096, 288
TC = 128
WORDS_PER_ROW = F_BINS // 4
PAIR_WORDS = 2 * WORDS_PER_ROW
NUM_PAIRS = TC // 2


def _winner_table(midi_notes):
    bins = jnp.clip(3 * midi_notes.astype(jnp.int32) - 63, 0, F_BINS - 1)
    key = (jnp.broadcast_to(bins[None, :] * T, (T, N))
           + jnp.arange(T, dtype=jnp.int32)[:, None])
    val = jnp.broadcast_to(jnp.arange(N, dtype=jnp.int32)[None, :], (T, N))
    _, vv = lax.sort((key.reshape(-1), val.reshape(-1)), dimension=0,
                     num_keys=1, is_stable=False)
    c = jnp.zeros((F_BINS,), jnp.int32).at[bins].add(1)
    cumex = jnp.cumsum(c) - c
    pos0 = (cumex[:, None] * T + c[:, None] * (jnp.arange(T, dtype=jnp.int32)[None, :] + 1)
            - 1)
    w = jnp.where(c[:, None] > 0, jnp.take(vv, pos0), -1)
    return w.T.reshape(-1)


def _sc_body(y_hbm, w_hbm, out_hbm, wv, yv, ov):
    wid = lax.axis_index("s") * 2 + lax.axis_index("c")
    t0 = wid * TC
    pltpu.sync_copy(w_hbm.at[pl.ds(t0 * F_BINS, TC * F_BINS)], wv)
    lane = lax.iota(jnp.int32, 16)
    for b in range(B):
        pltpu.sync_copy(y_hbm.at[b, :, pl.ds(t0, TC)], yv)

        def pair_body(pr, carry):
            base = pr * PAIR_WORDS
            for g in range(9):
                word = jnp.zeros((16,), jnp.int32)
                for kb in range(4):
                    o = g * 64 + kb + 4 * lane
                    toff = jnp.where(o >= F_BINS, 1, 0)
                    fvec = o - F_BINS * toff
                    trel = 2 * pr + toff
                    wg = plsc.load_gather(wv, [trel * F_BINS + fvec])
                    yval = plsc.load_gather(yv, [jnp.maximum(wg, 0), trel])
                    bit = jnp.where((wg >= 0) & (yval != 0.0), 1, 0)
                    word = word | (bit << (8 * kb))
                ov[pl.ds(base + g * 16, 16)] = word
            return carry

        lax.fori_loop(0, NUM_PAIRS, pair_body, 0)
        pltpu.sync_copy(ov, out_hbm.at[b, pl.ds(t0 * WORDS_PER_ROW, TC * WORDS_PER_ROW)])


@functools.partial(
    pl.kernel,
    mesh=plsc.VectorSubcoreMesh(core_axis_name="c", subcore_axis_name="s"),
    out_type=jax.ShapeDtypeStruct((B, T * WORDS_PER_ROW), jnp.int32),
    compiler_params=pltpu.CompilerParams(needs_layout_passes=False),
    scratch_types=[
        pltpu.VMEM((TC * F_BINS,), jnp.int32),
        pltpu.VMEM((N, TC), jnp.float32),
        pltpu.VMEM((TC * WORDS_PER_ROW,), jnp.int32),
    ],
)
def _sc_gather(y_hbm, w_hbm, out_hbm, wv, yv, ov):
    _sc_body(y_hbm, w_hbm, out_hbm, wv, yv, ov)


def kernel(y, midi_notes, F):
    del F
    w_flat = _winner_table(midi_notes)
    words = _sc_gather(y, w_flat)
    by = lax.bitcast_convert_type(words.reshape(B, T, WORDS_PER_ROW), jnp.uint8)
    return by.reshape(B, T, F_BINS) != 0
